# baseline (speedup 1.0000x reference)
import functools
import jax
import jax.numpy as jnp
import numpy as np
from jax import lax
from jax.experimental import pallas as pl
from jax.experimental.pallas import tpu as pltpu
from jax.experimental.pallas import tpu_sc as plsc

N = 10000
E = 320000
K = 25
KPAD = 28
NP = 10240

LAYER_GROUP = {16: 4, 32: 2, 64: 1}


_EBLK = 2000
E2 = E + _EBLK
_NWBLK = E2 // _EBLK


def _w_body(maps_ref, c_ref, w_ref):
    m = maps_ref[...]
    c = c_ref[...]
    m0 = m[:, 0:1]
    m1 = m[:, 1:2]
    one = jnp.ones_like(m0)
    z = jnp.zeros_like(m0)
    f = jnp.concatenate([m0 * m0, m0, m1 * m1, m1, one, z, z, z], axis=1)
    val = jnp.exp(jnp.dot(f, c, preferred_element_type=jnp.float32))
    w_ref[...] = jnp.where(pl.program_id(0) == _NWBLK - 1, 0.0, val)


def _edge_weights(maps, mu, sigma):
    inv = -0.5 / (sigma * sigma + 1e-14)
    a = inv[:, 0]
    b = -2.0 * inv[:, 0] * mu[:, 0]
    c = inv[:, 1]
    d = -2.0 * inv[:, 1] * mu[:, 1]
    e = inv[:, 0] * mu[:, 0] ** 2 + inv[:, 1] * mu[:, 1] ** 2
    coeff = jnp.stack([a, b, c, d, e], axis=0)
    coeff = jnp.pad(coeff, ((0, 3), (0, 0)))
    pad = jnp.full((8, 32 - K), 0.0, jnp.float32).at[4, :].set(-1e30)
    coeff = jnp.concatenate([coeff, pad], axis=1)
    return pl.pallas_call(
        _w_body,
        grid=(_NWBLK,),
        in_specs=[
            pl.BlockSpec((_EBLK, 2), lambda i: (jnp.minimum(i, E // _EBLK - 1), 0)),
            pl.BlockSpec((8, 32), lambda i: (0, 0)),
        ],
        out_specs=pl.BlockSpec((_EBLK, 32), lambda i: (i, 0)),
        out_shape=jax.ShapeDtypeStruct((E2, 32), jnp.float32),
    )(maps, coeff)


def _elu(a):
    return jnp.where(a > 0, a, jnp.exp(jnp.minimum(a, 0.0)) - 1.0)


def _dense_body(x_ref, w_ref, b_ref, o_ref, *, act):
    r = jnp.dot(x_ref[...], w_ref[...], preferred_element_type=jnp.float32) + b_ref[...]
    o_ref[...] = _elu(r) if act else r


def _dense(x, W, b, act=True, blk=1024):
    n, ci = x.shape
    co = W.shape[1]
    return pl.pallas_call(
        functools.partial(_dense_body, act=act),
        grid=(n // blk,),
        in_specs=[
            pl.BlockSpec((blk, ci), lambda i: (i, 0)),
            pl.BlockSpec((ci, co), lambda i: (0, 0)),
            pl.BlockSpec((1, co), lambda i: (0, 0)),
        ],
        out_specs=pl.BlockSpec((blk, co), lambda i: (i, 0)),
        out_shape=jax.ShapeDtypeStruct((n, co), jnp.float32),
    )(x, W, b.reshape(1, co))


def _post_body(t_ref, ws_ref, x_ref, root_ref, deg_ref, b_ref, o_ref):
    acc = jnp.dot(t_ref[...], ws_ref[...], preferred_element_type=jnp.float32)
    d = jnp.maximum(deg_ref[...], 1.0)
    r = acc / d + jnp.dot(x_ref[...], root_ref[...], preferred_element_type=jnp.float32) + b_ref[...]
    o_ref[...] = _elu(r)


def _post(T, Wstack, x, rootW, deg2, bias, blk=512):
    n, kpci = T.shape
    ci = x.shape[1]
    co = rootW.shape[1]
    return pl.pallas_call(
        _post_body,
        grid=(n // blk,),
        in_specs=[
            pl.BlockSpec((blk, kpci), lambda i: (i, 0)),
            pl.BlockSpec((kpci, co), lambda i: (0, 0)),
            pl.BlockSpec((blk, ci), lambda i: (i, 0)),
            pl.BlockSpec((ci, co), lambda i: (0, 0)),
            pl.BlockSpec((blk, 1), lambda i: (i, 0)),
            pl.BlockSpec((1, co), lambda i: (0, 0)),
        ],
        out_specs=pl.BlockSpec((blk, co), lambda i: (i, 0)),
        out_shape=jax.ShapeDtypeStruct((n, co), jnp.float32),
    )(T, Wstack, x, rootW, deg2, bias.reshape(1, co))


def _make_wstack(gW, ci, co):
    w3 = gW.reshape(ci, K, co).transpose(1, 0, 2)
    w3 = jnp.pad(w3, ((0, KPAD - K), (0, 0), (0, 0)))
    return w3.reshape(KPAD * ci, co)


def _head_body(h_ref, w1_ref, b1_ref, w2_ref, b2_ref, o_ref):
    l1 = _elu(jnp.dot(h_ref[...], w1_ref[...], preferred_element_type=jnp.float32) + b1_ref[...])
    l2 = jnp.dot(l1, w2_ref[...], preferred_element_type=jnp.float32) + b2_ref[...]
    m = jnp.max(l2, axis=1, keepdims=True)
    s = jnp.log(jnp.sum(jnp.exp(l2 - m), axis=1, keepdims=True))
    o_ref[...] = l2 - m - s


def _head(h, fc1_W, fc1_b, fc2_W, fc2_b, blk=1024):
    n = h.shape[0]
    nc = fc2_W.shape[1]
    w2 = jnp.pad(fc2_W, ((0, 0), (0, 128 - nc)))
    b2 = jnp.pad(fc2_b, (0, 128 - nc), constant_values=-1e30)
    return pl.pallas_call(
        _head_body,
        grid=(n // blk,),
        in_specs=[
            pl.BlockSpec((blk, 128), lambda i: (i, 0)),
            pl.BlockSpec((128, 256), lambda i: (0, 0)),
            pl.BlockSpec((1, 256), lambda i: (0, 0)),
            pl.BlockSpec((256, 128), lambda i: (0, 0)),
            pl.BlockSpec((1, 128), lambda i: (0, 0)),
        ],
        out_specs=pl.BlockSpec((blk, 128), lambda i: (i, 0)),
        out_shape=jax.ShapeDtypeStruct((n, 128), jnp.float32),
    )(h, fc1_W, fc1_b.reshape(1, 256), w2, b2.reshape(1, 128))


_NC, _NS = 2, 16
NW = _NC * _NS
ROWS = NP // NW
NB = 8
BROWS = ROWS // NB
CHUNK = 8000
FB = 512
FB2 = 128
CAPW = E + FB
CAP2 = E + NB * FB2
RPAD1 = 336

_mesh = plsc.VectorSubcoreMesh(core_axis_name="c", subcore_axis_name="s")


def _wid():
    return lax.axis_index("s") * _NC + lax.axis_index("c")


def _route_body(dst_hbm, src_hbm, ids_hbm, srcl_hbm, dll_hbm,
                ids2_hbm, srcl2_hbm, dll2_hbm, cnt_hbm, deg_hbm,
                dstb, srcb, ido, srco, dlo, p1i, p1s, p1d, degb, cntb):
    wid = _wid()
    base = wid * ROWS
    iota = lax.iota(jnp.int32, 16)
    zero16 = jnp.zeros((16,), jnp.float32)

    def zb(i, c):
        degb[pl.ds(i * 16, 16)] = zero16
        return c
    lax.fori_loop(0, RPAD1 // 16, zb, 0)

    def chunk_body(ch, carry):
        pltpu.sync_copy(dst_hbm.at[pl.ds(ch * CHUNK, CHUNK)], dstb)
        pltpu.sync_copy(src_hbm.at[pl.ds(ch * CHUNK, CHUNK)], srcb)

        def step(s, carry2):
            off, nblk = carry2
            dv = dstb[pl.ds(s * 16, 16)]
            m = (dv >= base) & (dv < base + ROWS)
            cnt = plsc.all_reduce_population_count(m)[0]
            ids = ch * CHUNK + s * 16 + iota
            plsc.store_compressed(ido.at[pl.ds(off, 16)], ids, mask=m)
            plsc.store_compressed(srco.at[pl.ds(off, 16)], srcb[pl.ds(s * 16, 16)], mask=m)
            plsc.store_compressed(dlo.at[pl.ds(off, 16)], dv - base, mask=m)
            off = off + cnt
            full = off >= FB

            @pl.when(full)
            def _():
                pltpu.sync_copy(ido.at[pl.ds(0, FB)], ids_hbm.at[wid, pl.ds(nblk * FB, FB)])
                pltpu.sync_copy(srco.at[pl.ds(0, FB)], srcl_hbm.at[wid, pl.ds(nblk * FB, FB)])
                pltpu.sync_copy(dlo.at[pl.ds(0, FB)], dll_hbm.at[wid, pl.ds(nblk * FB, FB)])
                ido[pl.ds(0, 16)] = ido[pl.ds(FB, 16)]
                srco[pl.ds(0, 16)] = srco[pl.ds(FB, 16)]
                dlo[pl.ds(0, 16)] = dlo[pl.ds(FB, 16)]

            off = jnp.where(full, off - FB, off)
            nblk = jnp.where(full, nblk + 1, nblk)
            return off, nblk

        return lax.fori_loop(0, CHUNK // 16, step, carry, unroll=4)

    off, nblk = lax.fori_loop(0, E // CHUNK, chunk_body, (0, 0))

    def fill(s, c):
        pos = s * 16 + iota
        mf = pos >= off
        ido[pl.ds(s * 16, 16)] = jnp.where(mf, 0, ido[pl.ds(s * 16, 16)])
        srco[pl.ds(s * 16, 16)] = jnp.where(mf, 0, srco[pl.ds(s * 16, 16)])
        dlo[pl.ds(s * 16, 16)] = jnp.where(mf, ROWS, dlo[pl.ds(s * 16, 16)])
        return c
    lax.fori_loop(0, FB // 16, fill, 0)

    @pl.when(off > 0)
    def _():
        pltpu.sync_copy(ido.at[pl.ds(0, FB)], ids_hbm.at[wid, pl.ds(nblk * FB, FB)])
        pltpu.sync_copy(srco.at[pl.ds(0, FB)], srcl_hbm.at[wid, pl.ds(nblk * FB, FB)])
        pltpu.sync_copy(dlo.at[pl.ds(0, FB)], dll_hbm.at[wid, pl.ds(nblk * FB, FB)])

    nblk = jnp.where(off > 0, nblk + 1, nblk)

    onehot0 = jnp.where(iota == 0, 1.0, 0.0).astype(jnp.float32)

    def degblk(b, c):
        pltpu.sync_copy(dll_hbm.at[wid, pl.ds(b * FB, FB)], p1d.at[pl.ds(0, FB)])

        def de(e, c2):
            dl = p1d[pl.ds(e, 16)][0]
            v = degb[pl.ds(dl, 16)]
            degb[pl.ds(dl, 16)] = v + onehot0
            return c2
        return lax.fori_loop(0, FB, de, c)
    lax.fori_loop(0, nblk, degblk, 0)
    pltpu.sync_copy(degb.at[pl.ds(0, ROWS)], deg_hbm.at[pl.ds(wid * ROWS, ROWS)])

    def bucket(sb, carry):
        gblk0 = carry
        lo = sb * BROWS

        def bblk(b, carry2):
            pltpu.sync_copy(ids_hbm.at[wid, pl.ds(b * FB, FB)], p1i.at[pl.ds(0, FB)])
            pltpu.sync_copy(srcl_hbm.at[wid, pl.ds(b * FB, FB)], p1s.at[pl.ds(0, FB)])
            pltpu.sync_copy(dll_hbm.at[wid, pl.ds(b * FB, FB)], p1d.at[pl.ds(0, FB)])

            def st(s2, carry3):
                off2, gblk = carry3
                dv = p1d[pl.ds(s2 * 16, 16)]
                m = (dv >= lo) & (dv < lo + BROWS)
                cnt = plsc.all_reduce_population_count(m)[0]
                plsc.store_compressed(ido.at[pl.ds(off2, 16)], p1i[pl.ds(s2 * 16, 16)], mask=m)
                plsc.store_compressed(srco.at[pl.ds(off2, 16)], p1s[pl.ds(s2 * 16, 16)], mask=m)
                plsc.store_compressed(dlo.at[pl.ds(off2, 16)], dv, mask=m)
                off2 = off2 + cnt
                full = off2 >= FB2

                @pl.when(full)
                def _():
                    pltpu.sync_copy(ido.at[pl.ds(0, FB2)], ids2_hbm.at[wid, pl.ds(gblk * FB2, FB2)])
                    pltpu.sync_copy(srco.at[pl.ds(0, FB2)], srcl2_hbm.at[wid, pl.ds(gblk * FB2, FB2)])
                    pltpu.sync_copy(dlo.at[pl.ds(0, FB2)], dll2_hbm.at[wid, pl.ds(gblk * FB2, FB2)])
                    ido[pl.ds(0, 16)] = ido[pl.ds(FB2, 16)]
                    srco[pl.ds(0, 16)] = srco[pl.ds(FB2, 16)]
                    dlo[pl.ds(0, 16)] = dlo[pl.ds(FB2, 16)]

                off2 = jnp.where(full, off2 - FB2, off2)
                gblk = jnp.where(full, gblk + 1, gblk)
                return off2, gblk

            return lax.fori_loop(0, FB // 16, st, carry2, unroll=4)

        off2, gblk = lax.fori_loop(0, nblk, bblk, (0, gblk0))

        def fill2(s, c):
            pos = s * 16 + iota
            mf = pos >= off2
            ido[pl.ds(s * 16, 16)] = jnp.where(mf, E, ido[pl.ds(s * 16, 16)])
            srco[pl.ds(s * 16, 16)] = jnp.where(mf, 0, srco[pl.ds(s * 16, 16)])
            dlo[pl.ds(s * 16, 16)] = jnp.where(mf, lo + BROWS, dlo[pl.ds(s * 16, 16)])
            return c
        lax.fori_loop(0, FB2 // 16, fill2, 0)

        @pl.when(off2 > 0)
        def _():
            pltpu.sync_copy(ido.at[pl.ds(0, FB2)], ids2_hbm.at[wid, pl.ds(gblk * FB2, FB2)])
            pltpu.sync_copy(srco.at[pl.ds(0, FB2)], srcl2_hbm.at[wid, pl.ds(gblk * FB2, FB2)])
            pltpu.sync_copy(dlo.at[pl.ds(0, FB2)], dll2_hbm.at[wid, pl.ds(gblk * FB2, FB2)])

        gblk = jnp.where(off2 > 0, gblk + 1, gblk)
        cv = cntb[...]
        cntb[...] = jnp.where(iota == sb, gblk, cv)
        return gblk

    cntb[...] = jnp.zeros((16,), jnp.int32)
    lax.fori_loop(0, NB, bucket, 0)
    pltpu.sync_copy(cntb, cnt_hbm.at[wid])


@functools.partial(
    pl.kernel, mesh=_mesh,
    compiler_params=pltpu.CompilerParams(needs_layout_passes=False, use_tc_tiling_on_sc=False),
    out_type=(
        jax.ShapeDtypeStruct((NW, CAPW), jnp.int32),
        jax.ShapeDtypeStruct((NW, CAPW), jnp.int32),
        jax.ShapeDtypeStruct((NW, CAPW), jnp.int32),
        jax.ShapeDtypeStruct((NW, CAP2), jnp.int32),
        jax.ShapeDtypeStruct((NW, CAP2), jnp.int32),
        jax.ShapeDtypeStruct((NW, CAP2), jnp.int32),
        jax.ShapeDtypeStruct((NW, 16), jnp.int32),
        jax.ShapeDtypeStruct((NP,), jnp.float32),
    ),
    scratch_types=[
        pltpu.VMEM((CHUNK,), jnp.int32),
        pltpu.VMEM((CHUNK,), jnp.int32),
        pltpu.VMEM((FB + 16,), jnp.int32),
        pltpu.VMEM((FB + 16,), jnp.int32),
        pltpu.VMEM((FB + 16,), jnp.int32),
        pltpu.VMEM((FB + 16,), jnp.int32),
        pltpu.VMEM((FB + 16,), jnp.int32),
        pltpu.VMEM((FB + 16,), jnp.int32),
        pltpu.VMEM((RPAD1 + 16,), jnp.float32),
        pltpu.VMEM((16,), jnp.int32),
    ],
)
def _route(*refs):
    _route_body(*refs)


def _agg_body(x_hbm, w_hbm, ids_hbm, srcl_hbm, dll_hbm, cnt_hbm, t_hbm,
              idb0, srcb0, dlb0, xb0, wb0, idb1, srcb1, dlb1, xb1, wb1,
              accb, cntb, semx0, semw0, semx1, semw1, *, ci, g):
    wid = _wid()
    kpci = KPAD * ci
    nvec = ci // 16
    npass = NB // g
    arows = g * BROWS + 8
    pltpu.sync_copy(cnt_hbm.at[wid], cntb)
    ends = cntb[...]
    zero16 = jnp.zeros((16,), jnp.float32)
    slots = ((idb0, srcb0, dlb0, xb0, wb0, semx0, semw0),
             (idb1, srcb1, dlb1, xb1, wb1, semx1, semw1))

    def load_fire(b, slot):
        idb, srcb, dlb, xb, wb, semx, semw = slots[slot]
        pltpu.sync_copy(ids_hbm.at[wid, pl.ds(b * FB2, FB2)], idb)
        pltpu.sync_copy(srcl_hbm.at[wid, pl.ds(b * FB2, FB2)], srcb)
        pltpu.sync_copy(dll_hbm.at[wid, pl.ds(b * FB2, FB2)], dlb.at[pl.ds(0, FB2)])
        pltpu.async_copy(x_hbm.at[srcb], xb, semx)
        pltpu.async_copy(w_hbm.at[idb], wb, semw)

    def wait(slot):
        idb, srcb, dlb, xb, wb, semx, semw = slots[slot]
        pltpu.make_async_copy(x_hbm.at[srcb], xb, semx).wait()
        pltpu.make_async_copy(w_hbm.at[idb], wb, semw).wait()

    def pass_body(p, cpass):
        base_p = p * g * BROWS

        def za(i, c):
            accb[pl.ds(i * 16, 16)] = zero16
            return c
        lax.fori_loop(0, arows * kpci // 16, za, 0)

        def compute(slot, base_p):
            idb, srcb, dlb, xb, wb, semx, semw = slots[slot]

            def edge(e, c2):
                dl = dlb[pl.ds(e, 16)][0]
                rb = (dl - base_p) * kpci
                xv = [xb[e, pl.ds(cc * 16, 16)] for cc in range(nvec)]
                wr0 = wb[e, pl.ds(0, 16)]
                wr1 = wb[e, pl.ds(16, 16)]
                for j in range(KPAD):
                    wk = wr0[j] if j < 16 else wr1[j - 16]
                    for cc in range(nvec):
                        plsc.addupdate(accb.at[pl.ds(rb + j * ci + cc * 16, 16)], wk * xv[cc])
                return c2
            lax.fori_loop(0, FB2, edge, 0, unroll=2)

        for sq in range(g):
            s = p * g + sq
            sv = jnp.full((16,), s, jnp.int32)
            b_hi = plsc.load_gather(cntb, [sv])[0]
            b_lom = plsc.load_gather(cntb, [jnp.maximum(sv - 1, 0)])[0]
            b_lo = jnp.where(s == 0, 0, b_lom)
            nb = b_hi - b_lo

            @pl.when(nb > 0)
            def _():
                load_fire(b_lo, 0)

            def pair(i, c):
                b0 = b_lo + 2 * i

                @pl.when(b0 + 1 < b_hi)
                def _():
                    load_fire(b0 + 1, 1)
                wait(0)
                compute(0, base_p)

                @pl.when(b0 + 2 < b_hi)
                def _():
                    load_fire(b0 + 2, 0)

                @pl.when(b0 + 1 < b_hi)
                def _():
                    wait(1)
                    compute(1, base_p)
                return c
            lax.fori_loop(0, (nb + 1) // 2, pair, 0)

        pltpu.sync_copy(accb.at[pl.ds(0, g * BROWS * kpci)],
                        t_hbm.at[pl.ds((wid * ROWS + base_p) * kpci, g * BROWS * kpci)])
        return cpass

    lax.fori_loop(0, npass, pass_body, 0)


def _make_agg(ci, g):
    kpci = KPAD * ci
    arows = g * BROWS + 8

    @functools.partial(
        pl.kernel, mesh=_mesh,
        compiler_params=pltpu.CompilerParams(needs_layout_passes=False, use_tc_tiling_on_sc=False),
        out_type=jax.ShapeDtypeStruct((NP * kpci,), jnp.float32),
        scratch_types=[
            pltpu.VMEM((FB2,), jnp.int32),
            pltpu.VMEM((FB2,), jnp.int32),
            pltpu.VMEM((FB2 + 16,), jnp.int32),
            pltpu.VMEM((FB2, ci), jnp.float32),
            pltpu.VMEM((FB2, 32), jnp.float32),
            pltpu.VMEM((FB2,), jnp.int32),
            pltpu.VMEM((FB2,), jnp.int32),
            pltpu.VMEM((FB2 + 16,), jnp.int32),
            pltpu.VMEM((FB2, ci), jnp.float32),
            pltpu.VMEM((FB2, 32), jnp.float32),
            pltpu.VMEM((arows * kpci,), jnp.float32),
            pltpu.VMEM((16,), jnp.int32),
            pltpu.SemaphoreType.DMA,
            pltpu.SemaphoreType.DMA,
            pltpu.SemaphoreType.DMA,
            pltpu.SemaphoreType.DMA,
        ],
    )
    def agg(x_hbm, w_hbm, ids_hbm, srcl_hbm, dll_hbm, cnt_hbm, t_hbm, *scratch):
        _agg_body(x_hbm, w_hbm, ids_hbm, srcl_hbm, dll_hbm, cnt_hbm, t_hbm,
                  *scratch, ci=ci, g=g)

    return agg


_AGG = {ci: _make_agg(ci, LAYER_GROUP[ci]) for ci in (16, 32, 64)}


def kernel(x, neighbours, maps, fc0_W, fc0_b, g1_W, mu1, sigma1, root1_W, bias1, g2_W, mu2, sigma2, root2_W, bias2, g3_W, mu3, sigma3, root3_W, bias3, fc1_W, fc1_b, fc2_W, fc2_b):
    src = neighbours[0]
    dst = neighbours[1]
    _, _, _, ids2, src2, dll2, cnt2, deg = _route(dst, src)
    deg2 = deg.reshape(NP, 1)

    w1 = _edge_weights(maps, mu1, sigma1)
    w2 = _edge_weights(maps, mu2, sigma2)
    w3 = _edge_weights(maps, mu3, sigma3)

    xp = jnp.pad(x, ((0, NP - N), (0, 5)))
    W0 = jnp.pad(fc0_W, ((0, 5), (0, 0)))
    h = _dense(xp, W0, fc0_b)

    for (wl, gW, rootW, bias, ci, co) in (
        (w1, g1_W, root1_W, bias1, 16, 32),
        (w2, g2_W, root2_W, bias2, 32, 64),
        (w3, g3_W, root3_W, bias3, 64, 128),
    ):
        t_flat = _AGG[ci](h, wl, ids2, src2, dll2, cnt2)
        T = t_flat.reshape(NP, KPAD * ci)
        Wstack = _make_wstack(gW, ci, co)
        h = _post(T, Wstack, h, rootW, deg2, bias)

    out = _head(h, fc1_W, fc1_b, fc2_W, fc2_b)
    return out[:N, :40]


# speedup vs baseline: 1.0217x; 1.0217x over previous
import functools
import jax
import jax.numpy as jnp
import numpy as np
from jax import lax
from jax.experimental import pallas as pl
from jax.experimental.pallas import tpu as pltpu
from jax.experimental.pallas import tpu_sc as plsc

N = 10000
E = 320000
K = 25
KPAD = 28
NP = 10240

LAYER_GROUP = {16: 4, 32: 2, 64: 1}


_EBLK = 2000
E2 = E + _EBLK
_NWBLK = E2 // _EBLK


def _w_body(maps_ref, c_ref, w_ref):
    m = maps_ref[...]
    c = c_ref[...]
    m0 = m[:, 0:1]
    m1 = m[:, 1:2]
    one = jnp.ones_like(m0)
    z = jnp.zeros_like(m0)
    f = jnp.concatenate([m0 * m0, m0, m1 * m1, m1, one, z, z, z], axis=1)
    val = jnp.exp(jnp.dot(f, c, preferred_element_type=jnp.float32))
    w_ref[...] = jnp.where(pl.program_id(0) == _NWBLK - 1, 0.0, val)


def _edge_weights(maps, mu, sigma):
    inv = -0.5 / (sigma * sigma + 1e-14)
    a = inv[:, 0]
    b = -2.0 * inv[:, 0] * mu[:, 0]
    c = inv[:, 1]
    d = -2.0 * inv[:, 1] * mu[:, 1]
    e = inv[:, 0] * mu[:, 0] ** 2 + inv[:, 1] * mu[:, 1] ** 2
    coeff = jnp.stack([a, b, c, d, e], axis=0)
    coeff = jnp.pad(coeff, ((0, 3), (0, 0)))
    pad = jnp.full((8, 32 - K), 0.0, jnp.float32).at[4, :].set(-1e30)
    coeff = jnp.concatenate([coeff, pad], axis=1)
    return pl.pallas_call(
        _w_body,
        grid=(_NWBLK,),
        in_specs=[
            pl.BlockSpec((_EBLK, 2), lambda i: (jnp.minimum(i, E // _EBLK - 1), 0)),
            pl.BlockSpec((8, 32), lambda i: (0, 0)),
        ],
        out_specs=pl.BlockSpec((_EBLK, 32), lambda i: (i, 0)),
        out_shape=jax.ShapeDtypeStruct((E2, 32), jnp.float32),
    )(maps, coeff)


def _elu(a):
    return jnp.where(a > 0, a, jnp.exp(jnp.minimum(a, 0.0)) - 1.0)


def _dense_body(x_ref, w_ref, b_ref, o_ref, *, act):
    r = jnp.dot(x_ref[...], w_ref[...], preferred_element_type=jnp.float32) + b_ref[...]
    o_ref[...] = _elu(r) if act else r


def _dense(x, W, b, act=True, blk=1024):
    n, ci = x.shape
    co = W.shape[1]
    return pl.pallas_call(
        functools.partial(_dense_body, act=act),
        grid=(n // blk,),
        in_specs=[
            pl.BlockSpec((blk, ci), lambda i: (i, 0)),
            pl.BlockSpec((ci, co), lambda i: (0, 0)),
            pl.BlockSpec((1, co), lambda i: (0, 0)),
        ],
        out_specs=pl.BlockSpec((blk, co), lambda i: (i, 0)),
        out_shape=jax.ShapeDtypeStruct((n, co), jnp.float32),
    )(x, W, b.reshape(1, co))


def _post_body(t_ref, ws_ref, x_ref, root_ref, deg_ref, b_ref, o_ref):
    acc = jnp.dot(t_ref[...], ws_ref[...], preferred_element_type=jnp.float32)
    d = jnp.maximum(deg_ref[...], 1.0)
    r = acc / d + jnp.dot(x_ref[...], root_ref[...], preferred_element_type=jnp.float32) + b_ref[...]
    o_ref[...] = _elu(r)


def _post(T, Wstack, x, rootW, deg2, bias, blk=512):
    n, kpci = T.shape
    ci = x.shape[1]
    co = rootW.shape[1]
    return pl.pallas_call(
        _post_body,
        grid=(n // blk,),
        in_specs=[
            pl.BlockSpec((blk, kpci), lambda i: (i, 0)),
            pl.BlockSpec((kpci, co), lambda i: (0, 0)),
            pl.BlockSpec((blk, ci), lambda i: (i, 0)),
            pl.BlockSpec((ci, co), lambda i: (0, 0)),
            pl.BlockSpec((blk, 1), lambda i: (i, 0)),
            pl.BlockSpec((1, co), lambda i: (0, 0)),
        ],
        out_specs=pl.BlockSpec((blk, co), lambda i: (i, 0)),
        out_shape=jax.ShapeDtypeStruct((n, co), jnp.float32),
    )(T, Wstack, x, rootW, deg2, bias.reshape(1, co))


def _make_wstack(gW, ci, co):
    w3 = gW.reshape(ci, K, co).transpose(1, 0, 2)
    w3 = jnp.pad(w3, ((0, KPAD - K), (0, 0), (0, 0)))
    return w3.reshape(KPAD * ci, co)


def _head_body(h_ref, w1_ref, b1_ref, w2_ref, b2_ref, o_ref):
    l1 = _elu(jnp.dot(h_ref[...], w1_ref[...], preferred_element_type=jnp.float32) + b1_ref[...])
    l2 = jnp.dot(l1, w2_ref[...], preferred_element_type=jnp.float32) + b2_ref[...]
    m = jnp.max(l2, axis=1, keepdims=True)
    s = jnp.log(jnp.sum(jnp.exp(l2 - m), axis=1, keepdims=True))
    o_ref[...] = l2 - m - s


def _head(h, fc1_W, fc1_b, fc2_W, fc2_b, blk=1024):
    n = h.shape[0]
    nc = fc2_W.shape[1]
    w2 = jnp.pad(fc2_W, ((0, 0), (0, 128 - nc)))
    b2 = jnp.pad(fc2_b, (0, 128 - nc), constant_values=-1e30)
    return pl.pallas_call(
        _head_body,
        grid=(n // blk,),
        in_specs=[
            pl.BlockSpec((blk, 128), lambda i: (i, 0)),
            pl.BlockSpec((128, 256), lambda i: (0, 0)),
            pl.BlockSpec((1, 256), lambda i: (0, 0)),
            pl.BlockSpec((256, 128), lambda i: (0, 0)),
            pl.BlockSpec((1, 128), lambda i: (0, 0)),
        ],
        out_specs=pl.BlockSpec((blk, 128), lambda i: (i, 0)),
        out_shape=jax.ShapeDtypeStruct((n, 128), jnp.float32),
    )(h, fc1_W, fc1_b.reshape(1, 256), w2, b2.reshape(1, 128))


_NC, _NS = 2, 16
NW = _NC * _NS
ROWS = NP // NW
NB = 8
BROWS = ROWS // NB
CHUNK = 8000
FB = 512
FB2 = 128
CAPW = E + FB
CAP2 = E + NB * FB2
RPAD1 = 336

@functools.lru_cache(None)
def _get_mesh():
    return plsc.VectorSubcoreMesh(core_axis_name="c", subcore_axis_name="s")


def _wid():
    return lax.axis_index("s") * _NC + lax.axis_index("c")


def _route_body(dst_hbm, src_hbm, ids_hbm, srcl_hbm, dll_hbm,
                ids2_hbm, srcl2_hbm, dll2_hbm, cnt_hbm, deg_hbm,
                dstb, srcb, ido, srco, dlo, cntarr, p1i, p1s, p1d, degb, cntb):
    wid = _wid()
    base = wid * ROWS
    iota = lax.iota(jnp.int32, 16)
    zero16 = jnp.zeros((16,), jnp.float32)

    def zb(i, c):
        degb[pl.ds(i * 16, 16)] = zero16
        return c
    lax.fori_loop(0, RPAD1 // 16, zb, 0)

    NSTEP = CHUNK // 16
    NGRP = (NSTEP + 15) // 16

    def chunk_body(ch, carry):
        carry_in, nblk = carry
        pltpu.sync_copy(dst_hbm.at[pl.ds(ch * CHUNK, CHUNK)], dstb)
        pltpu.sync_copy(src_hbm.at[pl.ds(ch * CHUNK, CHUNK)], srcb)

        plsc.store_scatter(cntarr, [(iota + NGRP * 16 - 16) * 16],
                           jnp.zeros((16,), jnp.int32))

        def stepA(s, c):
            dv = dstb[pl.ds(s * 16, 16)]
            m = (dv >= base) & (dv < base + ROWS)
            cntarr[pl.ds(s * 16, 16)] = plsc.all_reduce_population_count(m)
            return c
        lax.fori_loop(0, NSTEP, stepA, 0, unroll=4)

        def grp(j, coff):
            idx = iota * 16 + j * 256
            cv = plsc.load_gather(cntarr, [idx])
            inc = plsc.cumsum(cv)
            plsc.store_scatter(cntarr, [idx], inc - cv + coff)
            return coff + inc[15]
        total = lax.fori_loop(0, NGRP, grp, carry_in)

        def stepB(s, c):
            off = cntarr[pl.ds(s * 16, 16)][0]
            dv = dstb[pl.ds(s * 16, 16)]
            m = (dv >= base) & (dv < base + ROWS)
            ids = ch * CHUNK + s * 16 + iota
            plsc.store_compressed(ido.at[pl.ds(off, 16)], ids, mask=m)
            plsc.store_compressed(srco.at[pl.ds(off, 16)], srcb[pl.ds(s * 16, 16)], mask=m)
            plsc.store_compressed(dlo.at[pl.ds(off, 16)], dv - base, mask=m)
            return c
        lax.fori_loop(0, NSTEP, stepB, 0, unroll=4)

        nfull = total // FB

        def fl(f, c):
            pltpu.sync_copy(ido.at[pl.ds(f * FB, FB)], ids_hbm.at[wid, pl.ds((nblk + f) * FB, FB)])
            pltpu.sync_copy(srco.at[pl.ds(f * FB, FB)], srcl_hbm.at[wid, pl.ds((nblk + f) * FB, FB)])
            pltpu.sync_copy(dlo.at[pl.ds(f * FB, FB)], dll_hbm.at[wid, pl.ds((nblk + f) * FB, FB)])
            return c
        lax.fori_loop(0, nfull, fl, 0)

        @pl.when(nfull > 0)
        def _():
            def mv(j, c):
                ido[pl.ds(j * 16, 16)] = ido[pl.ds(nfull * FB + j * 16, 16)]
                srco[pl.ds(j * 16, 16)] = srco[pl.ds(nfull * FB + j * 16, 16)]
                dlo[pl.ds(j * 16, 16)] = dlo[pl.ds(nfull * FB + j * 16, 16)]
                return c
            lax.fori_loop(0, FB // 16, mv, 0)

        return total - nfull * FB, nblk + nfull

    off, nblk = lax.fori_loop(0, E // CHUNK, chunk_body, (0, 0))

    def fill(s, c):
        pos = s * 16 + iota
        mf = pos >= off
        ido[pl.ds(s * 16, 16)] = jnp.where(mf, 0, ido[pl.ds(s * 16, 16)])
        srco[pl.ds(s * 16, 16)] = jnp.where(mf, 0, srco[pl.ds(s * 16, 16)])
        dlo[pl.ds(s * 16, 16)] = jnp.where(mf, ROWS, dlo[pl.ds(s * 16, 16)])
        return c
    lax.fori_loop(0, FB // 16, fill, 0)

    @pl.when(off > 0)
    def _():
        pltpu.sync_copy(ido.at[pl.ds(0, FB)], ids_hbm.at[wid, pl.ds(nblk * FB, FB)])
        pltpu.sync_copy(srco.at[pl.ds(0, FB)], srcl_hbm.at[wid, pl.ds(nblk * FB, FB)])
        pltpu.sync_copy(dlo.at[pl.ds(0, FB)], dll_hbm.at[wid, pl.ds(nblk * FB, FB)])

    nblk = jnp.where(off > 0, nblk + 1, nblk)

    onehot0 = jnp.where(iota == 0, 1.0, 0.0).astype(jnp.float32)

    def degblk(b, c):
        pltpu.sync_copy(dll_hbm.at[wid, pl.ds(b * FB, FB)], p1d.at[pl.ds(0, FB)])

        def de(e, c2):
            dl = p1d[pl.ds(e, 16)][0]
            v = degb[pl.ds(dl, 16)]
            degb[pl.ds(dl, 16)] = v + onehot0
            return c2
        return lax.fori_loop(0, FB, de, c)
    lax.fori_loop(0, nblk, degblk, 0)
    pltpu.sync_copy(degb.at[pl.ds(0, ROWS)], deg_hbm.at[pl.ds(wid * ROWS, ROWS)])

    def bucket(sb, carry):
        gblk0 = carry
        lo = sb * BROWS

        def bblk(b, carry2):
            pltpu.sync_copy(ids_hbm.at[wid, pl.ds(b * FB, FB)], p1i.at[pl.ds(0, FB)])
            pltpu.sync_copy(srcl_hbm.at[wid, pl.ds(b * FB, FB)], p1s.at[pl.ds(0, FB)])
            pltpu.sync_copy(dll_hbm.at[wid, pl.ds(b * FB, FB)], p1d.at[pl.ds(0, FB)])

            def st(s2, carry3):
                off2, gblk = carry3
                dv = p1d[pl.ds(s2 * 16, 16)]
                m = (dv >= lo) & (dv < lo + BROWS)
                cnt = plsc.all_reduce_population_count(m)[0]
                plsc.store_compressed(ido.at[pl.ds(off2, 16)], p1i[pl.ds(s2 * 16, 16)], mask=m)
                plsc.store_compressed(srco.at[pl.ds(off2, 16)], p1s[pl.ds(s2 * 16, 16)], mask=m)
                plsc.store_compressed(dlo.at[pl.ds(off2, 16)], dv, mask=m)
                off2 = off2 + cnt
                full = off2 >= FB2

                @pl.when(full)
                def _():
                    pltpu.sync_copy(ido.at[pl.ds(0, FB2)], ids2_hbm.at[wid, pl.ds(gblk * FB2, FB2)])
                    pltpu.sync_copy(srco.at[pl.ds(0, FB2)], srcl2_hbm.at[wid, pl.ds(gblk * FB2, FB2)])
                    pltpu.sync_copy(dlo.at[pl.ds(0, FB2)], dll2_hbm.at[wid, pl.ds(gblk * FB2, FB2)])
                    ido[pl.ds(0, 16)] = ido[pl.ds(FB2, 16)]
                    srco[pl.ds(0, 16)] = srco[pl.ds(FB2, 16)]
                    dlo[pl.ds(0, 16)] = dlo[pl.ds(FB2, 16)]

                off2 = jnp.where(full, off2 - FB2, off2)
                gblk = jnp.where(full, gblk + 1, gblk)
                return off2, gblk

            return lax.fori_loop(0, FB // 16, st, carry2, unroll=4)

        off2, gblk = lax.fori_loop(0, nblk, bblk, (0, gblk0))

        def fill2(s, c):
            pos = s * 16 + iota
            mf = pos >= off2
            ido[pl.ds(s * 16, 16)] = jnp.where(mf, E, ido[pl.ds(s * 16, 16)])
            srco[pl.ds(s * 16, 16)] = jnp.where(mf, 0, srco[pl.ds(s * 16, 16)])
            dlo[pl.ds(s * 16, 16)] = jnp.where(mf, lo + BROWS, dlo[pl.ds(s * 16, 16)])
            return c
        lax.fori_loop(0, FB2 // 16, fill2, 0)

        @pl.when(off2 > 0)
        def _():
            pltpu.sync_copy(ido.at[pl.ds(0, FB2)], ids2_hbm.at[wid, pl.ds(gblk * FB2, FB2)])
            pltpu.sync_copy(srco.at[pl.ds(0, FB2)], srcl2_hbm.at[wid, pl.ds(gblk * FB2, FB2)])
            pltpu.sync_copy(dlo.at[pl.ds(0, FB2)], dll2_hbm.at[wid, pl.ds(gblk * FB2, FB2)])

        gblk = jnp.where(off2 > 0, gblk + 1, gblk)
        cv = cntb[...]
        cntb[...] = jnp.where(iota == sb, gblk, cv)
        return gblk

    cntb[...] = jnp.zeros((16,), jnp.int32)
    lax.fori_loop(0, NB, bucket, 0)
    pltpu.sync_copy(cntb, cnt_hbm.at[wid])


def _make_route():
  @functools.partial(
    pl.kernel, mesh=_get_mesh(),
    compiler_params=pltpu.CompilerParams(needs_layout_passes=False, use_tc_tiling_on_sc=False),
    out_type=(
        jax.ShapeDtypeStruct((NW, CAPW), jnp.int32),
        jax.ShapeDtypeStruct((NW, CAPW), jnp.int32),
        jax.ShapeDtypeStruct((NW, CAPW), jnp.int32),
        jax.ShapeDtypeStruct((NW, CAP2), jnp.int32),
        jax.ShapeDtypeStruct((NW, CAP2), jnp.int32),
        jax.ShapeDtypeStruct((NW, CAP2), jnp.int32),
        jax.ShapeDtypeStruct((NW, 16), jnp.int32),
        jax.ShapeDtypeStruct((NP,), jnp.float32),
    ),
    scratch_types=[
        pltpu.VMEM((CHUNK,), jnp.int32),
        pltpu.VMEM((CHUNK,), jnp.int32),
        pltpu.VMEM((CHUNK + 720,), jnp.int32),
        pltpu.VMEM((CHUNK + 720,), jnp.int32),
        pltpu.VMEM((CHUNK + 720,), jnp.int32),
        pltpu.VMEM((8192,), jnp.int32),
        pltpu.VMEM((FB + 16,), jnp.int32),
        pltpu.VMEM((FB + 16,), jnp.int32),
        pltpu.VMEM((FB + 16,), jnp.int32),
        pltpu.VMEM((RPAD1 + 16,), jnp.float32),
        pltpu.VMEM((16,), jnp.int32),
    ],
)
  def _route(*refs):
    _route_body(*refs)
  return _route


def _agg_body(x_hbm, w_hbm, ids_hbm, srcl_hbm, dll_hbm, cnt_hbm, t_hbm,
              idb0, srcb0, dlb0, xb0, wb0, idb1, srcb1, dlb1, xb1, wb1,
              accb, cntb, semx0, semw0, semx1, semw1, *, ci, g):
    wid = _wid()
    kpci = KPAD * ci
    nvec = ci // 16
    npass = NB // g
    arows = g * BROWS + 8
    pltpu.sync_copy(cnt_hbm.at[wid], cntb)
    ends = cntb[...]
    zero16 = jnp.zeros((16,), jnp.float32)
    slots = ((idb0, srcb0, dlb0, xb0, wb0, semx0, semw0),
             (idb1, srcb1, dlb1, xb1, wb1, semx1, semw1))

    def load_fire(b, slot):
        idb, srcb, dlb, xb, wb, semx, semw = slots[slot]
        pltpu.sync_copy(ids_hbm.at[wid, pl.ds(b * FB2, FB2)], idb)
        pltpu.sync_copy(srcl_hbm.at[wid, pl.ds(b * FB2, FB2)], srcb)
        pltpu.sync_copy(dll_hbm.at[wid, pl.ds(b * FB2, FB2)], dlb.at[pl.ds(0, FB2)])
        pltpu.async_copy(x_hbm.at[srcb], xb, semx)
        pltpu.async_copy(w_hbm.at[idb], wb, semw)

    def wait(slot):
        idb, srcb, dlb, xb, wb, semx, semw = slots[slot]
        pltpu.make_async_copy(x_hbm.at[srcb], xb, semx).wait()
        pltpu.make_async_copy(w_hbm.at[idb], wb, semw).wait()

    def pass_body(p, cpass):
        base_p = p * g * BROWS

        def za(i, c):
            accb[pl.ds(i * 16, 16)] = zero16
            return c
        lax.fori_loop(0, arows * kpci // 16, za, 0)

        def compute(slot, base_p):
            idb, srcb, dlb, xb, wb, semx, semw = slots[slot]

            def edge(e, c2):
                dl = dlb[pl.ds(e, 16)][0]
                rb = (dl - base_p) * kpci
                xv = [xb[e, pl.ds(cc * 16, 16)] for cc in range(nvec)]
                wr0 = wb[e, pl.ds(0, 16)]
                wr1 = wb[e, pl.ds(16, 16)]
                for j in range(KPAD):
                    wk = wr0[j] if j < 16 else wr1[j - 16]
                    for cc in range(nvec):
                        plsc.addupdate(accb.at[pl.ds(rb + j * ci + cc * 16, 16)], wk * xv[cc])
                return c2
            lax.fori_loop(0, FB2, edge, 0, unroll=2)

        for sq in range(g):
            s = p * g + sq
            sv = jnp.full((16,), s, jnp.int32)
            b_hi = plsc.load_gather(cntb, [sv])[0]
            b_lom = plsc.load_gather(cntb, [jnp.maximum(sv - 1, 0)])[0]
            b_lo = jnp.where(s == 0, 0, b_lom)
            nb = b_hi - b_lo

            @pl.when(nb > 0)
            def _():
                load_fire(b_lo, 0)

            def pair(i, c):
                b0 = b_lo + 2 * i

                @pl.when(b0 + 1 < b_hi)
                def _():
                    load_fire(b0 + 1, 1)
                wait(0)
                compute(0, base_p)

                @pl.when(b0 + 2 < b_hi)
                def _():
                    load_fire(b0 + 2, 0)

                @pl.when(b0 + 1 < b_hi)
                def _():
                    wait(1)
                    compute(1, base_p)
                return c
            lax.fori_loop(0, (nb + 1) // 2, pair, 0)

        pltpu.sync_copy(accb.at[pl.ds(0, g * BROWS * kpci)],
                        t_hbm.at[pl.ds((wid * ROWS + base_p) * kpci, g * BROWS * kpci)])
        return cpass

    lax.fori_loop(0, npass, pass_body, 0)


def _make_agg(ci, g):
    kpci = KPAD * ci
    arows = g * BROWS + 8

    @functools.partial(
        pl.kernel, mesh=_get_mesh(),
        compiler_params=pltpu.CompilerParams(needs_layout_passes=False, use_tc_tiling_on_sc=False),
        out_type=jax.ShapeDtypeStruct((NP * kpci,), jnp.float32),
        scratch_types=[
            pltpu.VMEM((FB2,), jnp.int32),
            pltpu.VMEM((FB2,), jnp.int32),
            pltpu.VMEM((FB2 + 16,), jnp.int32),
            pltpu.VMEM((FB2, ci), jnp.float32),
            pltpu.VMEM((FB2, 32), jnp.float32),
            pltpu.VMEM((FB2,), jnp.int32),
            pltpu.VMEM((FB2,), jnp.int32),
            pltpu.VMEM((FB2 + 16,), jnp.int32),
            pltpu.VMEM((FB2, ci), jnp.float32),
            pltpu.VMEM((FB2, 32), jnp.float32),
            pltpu.VMEM((arows * kpci,), jnp.float32),
            pltpu.VMEM((16,), jnp.int32),
            pltpu.SemaphoreType.DMA,
            pltpu.SemaphoreType.DMA,
            pltpu.SemaphoreType.DMA,
            pltpu.SemaphoreType.DMA,
        ],
    )
    def agg(x_hbm, w_hbm, ids_hbm, srcl_hbm, dll_hbm, cnt_hbm, t_hbm, *scratch):
        _agg_body(x_hbm, w_hbm, ids_hbm, srcl_hbm, dll_hbm, cnt_hbm, t_hbm,
                  *scratch, ci=ci, g=g)

    return agg


_make_route = functools.lru_cache(None)(_make_route)
_make_agg = functools.lru_cache(None)(_make_agg)


def kernel(x, neighbours, maps, fc0_W, fc0_b, g1_W, mu1, sigma1, root1_W, bias1, g2_W, mu2, sigma2, root2_W, bias2, g3_W, mu3, sigma3, root3_W, bias3, fc1_W, fc1_b, fc2_W, fc2_b):
    src = neighbours[0]
    dst = neighbours[1]
    _, _, _, ids2, src2, dll2, cnt2, deg = _make_route()(dst, src)
    deg2 = deg.reshape(NP, 1)

    w1 = _edge_weights(maps, mu1, sigma1)
    w2 = _edge_weights(maps, mu2, sigma2)
    w3 = _edge_weights(maps, mu3, sigma3)

    xp = jnp.pad(x, ((0, NP - N), (0, 5)))
    W0 = jnp.pad(fc0_W, ((0, 5), (0, 0)))
    h = _dense(xp, W0, fc0_b)

    for (wl, gW, rootW, bias, ci, co) in (
        (w1, g1_W, root1_W, bias1, 16, 32),
        (w2, g2_W, root2_W, bias2, 32, 64),
        (w3, g3_W, root3_W, bias3, 64, 128),
    ):
        t_flat = _make_agg(ci, LAYER_GROUP[ci])(h, wl, ids2, src2, dll2, cnt2)
        T = t_flat.reshape(NP, KPAD * ci)
        Wstack = _make_wstack(gW, ci, co)
        h = _post(T, Wstack, h, rootW, deg2, bias)

    out = _head(h, fc1_W, fc1_b, fc2_W, fc2_b)
    return out[:N, :40]
